# Initial kernel scaffold; baseline (speedup 1.0000x reference)
import jax
import jax.numpy as jnp
from jax.experimental import pallas as pl


def kernel(table, idx):
    raise NotImplementedError("write your pallas kernel here")


# speedup vs baseline: 1.8803x; 1.8803x over previous
import functools

import jax
import jax.numpy as jnp
from jax import lax
from jax.experimental import pallas as pl
from jax.experimental.pallas import tpu as pltpu
from jax.experimental.pallas import tpu_sc as plsc

_D = 128
_B = 1024
_P = 50
_R = _B * _P * 2
_NC, _NS = 2, 16
_NW = _NC * _NS
_C = 128
_PER_W = _R // _NW
_NCHUNK = _PER_W // _C

_mesh = plsc.VectorSubcoreMesh(core_axis_name="c", subcore_axis_name="s")


@functools.partial(
    pl.kernel,
    out_type=jax.ShapeDtypeStruct((_R, _D), jnp.float32),
    mesh=_mesh,
    scratch_types=[
        pltpu.VMEM((_NCHUNK, _C), jnp.int32),
        pltpu.VMEM((2, _C, _D), jnp.float32),
        pltpu.SemaphoreType.DMA,
    ],
)
def _gather(table_hbm, idx_hbm, out_hbm, idx_v, rows_v, gsem):
    wid = lax.axis_index("s") * _NC + lax.axis_index("c")
    base = wid * _PER_W
    pltpu.sync_copy(idx_hbm.at[wid], idx_v)
    pltpu.async_copy(table_hbm.at[idx_v.at[0]], rows_v.at[0], gsem)
    pltpu.async_copy(table_hbm.at[idx_v.at[1]], rows_v.at[1], gsem)

    @pl.loop(0, _NCHUNK)
    def _chunk(j):
        par = lax.rem(j, 2)
        pltpu.make_async_copy(
            table_hbm.at[idx_v.at[j]], rows_v.at[par], gsem
        ).wait()
        pltpu.sync_copy(rows_v.at[par], out_hbm.at[pl.ds(base + j * _C, _C)])

        @pl.when(j + 2 < _NCHUNK)
        def _start_next():
            pltpu.async_copy(table_hbm.at[idx_v.at[j + 2]], rows_v.at[par], gsem)


def kernel(table, idx):
    idx_flat = idx.reshape(_NW, _NCHUNK, _C)
    out = _gather(table, idx_flat)
    return out.reshape(_B, _P, 1, 2, _D)


# speedup vs baseline: 1.9088x; 1.0152x over previous
import functools

import jax
import jax.numpy as jnp
from jax import lax
from jax.experimental import pallas as pl
from jax.experimental.pallas import tpu as pltpu
from jax.experimental.pallas import tpu_sc as plsc

_D = 128
_B = 1024
_P = 50
_R = _B * _P * 2
_NC, _NS = 2, 16
_NW = _NC * _NS
_C = 128
_PER_W = _R // _NW
_NCHUNK = _PER_W // _C

_mesh = plsc.VectorSubcoreMesh(core_axis_name="c", subcore_axis_name="s")


@functools.partial(
    pl.kernel,
    out_type=jax.ShapeDtypeStruct((_R, _D), jnp.float32),
    mesh=_mesh,
    scratch_types=[
        pltpu.VMEM((_NCHUNK, _C), jnp.int32),
        pltpu.VMEM((4, _C, _D), jnp.float32),
        pltpu.SemaphoreType.DMA,
        pltpu.SemaphoreType.DMA,
    ],
)
def _gather(table_hbm, idx_hbm, out_hbm, idx_v, rows_v, gsem, osem):
    wid = lax.axis_index("s") * _NC + lax.axis_index("c")
    base = wid * _PER_W
    pltpu.sync_copy(idx_hbm.at[wid], idx_v)
    pltpu.async_copy(table_hbm.at[idx_v.at[0]], rows_v.at[0], gsem)
    pltpu.async_copy(table_hbm.at[idx_v.at[1]], rows_v.at[1], gsem)

    @pl.loop(0, _NCHUNK)
    def _chunk(j):
        buf = lax.rem(j, 4)
        pltpu.make_async_copy(
            table_hbm.at[idx_v.at[j]], rows_v.at[buf], gsem
        ).wait()
        pltpu.async_copy(
            rows_v.at[buf], out_hbm.at[pl.ds(base + j * _C, _C)], osem
        )

        @pl.when(j + 2 < _NCHUNK)
        def _start_next():
            nbuf = lax.rem(j + 2, 4)

            @pl.when(j >= 2)
            def _reclaim():
                pltpu.make_async_copy(
                    rows_v.at[nbuf],
                    out_hbm.at[pl.ds(base + (j - 2) * _C, _C)],
                    osem,
                ).wait()

            pltpu.async_copy(table_hbm.at[idx_v.at[j + 2]], rows_v.at[nbuf], gsem)

    for _ in range(4):
        pltpu.make_async_copy(
            rows_v.at[0], out_hbm.at[pl.ds(base, _C)], osem
        ).wait()


def kernel(table, idx):
    idx_flat = idx.reshape(_NW, _NCHUNK, _C)
    out = _gather(table, idx_flat)
    return out.reshape(_B, _P, 1, 2, _D)
